# baseline (speedup 1.0000x reference)
import functools

import jax
import jax.numpy as jnp
from jax import lax
from jax.experimental import pallas as pl
from jax.experimental.pallas import tpu as pltpu
from jax.experimental.pallas import tpu_sc as plsc


@functools.lru_cache(maxsize=None)
def _make_sc_lookup(V, D, T, B):
    info = plsc.get_sparse_core_info()
    NC, NS, L = info.num_cores, info.num_subcores, info.num_lanes
    NW = NC * NS
    assert D == L and B % (NW * L) == 0
    bw = B // NW
    n_h = bw // L

    mesh = plsc.VectorSubcoreMesh(core_axis_name="c", subcore_axis_name="s")

    @functools.partial(
        pl.kernel,
        out_type=jax.ShapeDtypeStruct((T, D, B), jnp.float32),
        mesh=mesh,
        compiler_params=pltpu.CompilerParams(
            use_tc_tiling_on_sc=False, needs_layout_passes=False
        ),
        scratch_types=[
            pltpu.VMEM((V * D,), jnp.float32),
            pltpu.VMEM((T, bw), jnp.int32),
            pltpu.VMEM((T, D, bw), jnp.float32),
        ],
    )
    def lookup(emb_hbm, ids_hbm, xt_hbm, emb_v, idx_v, tbuf):
        wid = lax.axis_index("s") * NC + lax.axis_index("c")
        bcol = wid * bw
        pltpu.sync_copy(emb_hbm, emb_v)
        pltpu.sync_copy(ids_hbm.at[:, pl.ds(bcol, bw)], idx_v)

        @plsc.parallel_loop(0, T, unroll=2)
        def tok(t):
            for h in range(n_h):
                idx_g = idx_v[t, pl.ds(h * L, L)]
                for d in range(D):
                    tbuf[t, d, pl.ds(h * L, L)] = plsc.load_gather(
                        emb_v, [idx_g * D + d]
                    )
        pltpu.sync_copy(tbuf, xt_hbm.at[:, :, pl.ds(bcol, bw)])

    return lookup


@functools.lru_cache(maxsize=None)
def _make_tc_head(T, BT, V, D, B):
    assert T % BT == 0

    def body(xt_ref, w_ref, b_ref, out_ref):
        for j in range(BT):
            out_ref[j] = (
                lax.dot_general(
                    w_ref[...], xt_ref[j],
                    dimension_numbers=(((0,), (0,)), ((), ())),
                    preferred_element_type=jnp.float32,
                )
                + b_ref[...]
            )

    return pl.pallas_call(
        body,
        grid=(T // BT,),
        in_specs=[
            pl.BlockSpec((BT, D, B), lambda i: (i, 0, 0)),
            pl.BlockSpec((D, V), lambda i: (0, 0)),
            pl.BlockSpec((V, 1), lambda i: (0, 0)),
        ],
        out_specs=pl.BlockSpec((BT, V, B), lambda i: (i, 0, 0)),
        out_shape=jax.ShapeDtypeStruct((T, V, B), jnp.float32),
    )


def kernel(input_ids, emb_table, W, b):
    Bsz, T = input_ids.shape
    V, D = emb_table.shape
    Vout = W.shape[1]

    ids_t = input_ids.astype(jnp.int32).T
    xt3 = _make_sc_lookup(V, D, T, Bsz)(emb_table.reshape(V * D), ids_t)
    tvb = _make_tc_head(T, 5, Vout, D, Bsz)(xt3, W, b.reshape(Vout, 1))
    return jnp.transpose(tvb, (2, 0, 1))


# speedup vs baseline: 1.0307x; 1.0307x over previous
import functools

import jax
import jax.numpy as jnp
from jax import lax
from jax.experimental import pallas as pl
from jax.experimental.pallas import tpu as pltpu
from jax.experimental.pallas import tpu_sc as plsc


@functools.lru_cache(maxsize=None)
def _make_sc_lookup(V, D, T, B):
    info = plsc.get_sparse_core_info()
    NC, NS, L = info.num_cores, info.num_subcores, info.num_lanes
    NW = NC * NS
    assert D == L and B % (NW * L) == 0
    bw = B // NW
    n_h = bw // L

    mesh = plsc.VectorSubcoreMesh(core_axis_name="c", subcore_axis_name="s")

    @functools.partial(
        pl.kernel,
        out_type=jax.ShapeDtypeStruct((T, D, B), jnp.float32),
        mesh=mesh,
        compiler_params=pltpu.CompilerParams(
            use_tc_tiling_on_sc=False, needs_layout_passes=False
        ),
        scratch_types=[
            pltpu.VMEM((V * D,), jnp.float32),
            pltpu.VMEM((T, bw), jnp.int32),
            pltpu.VMEM((T, D, bw), jnp.float32),
        ],
    )
    def lookup(emb_hbm, ids_hbm, xt_hbm, emb_v, idx_v, tbuf):
        wid = lax.axis_index("s") * NC + lax.axis_index("c")
        bcol = wid * bw
        pltpu.sync_copy(emb_hbm, emb_v)
        pltpu.sync_copy(ids_hbm.at[:, pl.ds(bcol, bw)], idx_v)

        @plsc.parallel_loop(0, T, unroll=2)
        def tok(t):
            for h in range(n_h):
                idx_g = idx_v[t, pl.ds(h * L, L)]
                for d in range(D):
                    tbuf[t, d, pl.ds(h * L, L)] = plsc.load_gather(
                        emb_v, [idx_g * D + d]
                    )
        pltpu.sync_copy(tbuf, xt_hbm.at[:, :, pl.ds(bcol, bw)])

    return lookup


@functools.lru_cache(maxsize=None)
def _make_tc_head(T, BT, V, D, B):
    assert T % BT == 0

    def body(xt_ref, w_ref, b_ref, out_ref):
        for j in range(BT):
            out_ref[j] = (
                lax.dot_general(
                    w_ref[...], xt_ref[j],
                    dimension_numbers=(((0,), (0,)), ((), ())),
                    preferred_element_type=jnp.float32,
                )
                + b_ref[...]
            )

    return pl.pallas_call(
        body,
        grid=(T // BT,),
        in_specs=[
            pl.BlockSpec((BT, D, B), lambda i: (i, 0, 0)),
            pl.BlockSpec((D, V), lambda i: (0, 0)),
            pl.BlockSpec((V, 1), lambda i: (0, 0)),
        ],
        out_specs=pl.BlockSpec((BT, V, B), lambda i: (i, 0, 0)),
        out_shape=jax.ShapeDtypeStruct((T, V, B), jnp.float32),
    )


def kernel(input_ids, emb_table, W, b):
    Bsz, T = input_ids.shape
    V, D = emb_table.shape
    Vout = W.shape[1]

    ids_t = input_ids.astype(jnp.int32).T
    xt3 = _make_sc_lookup(V, D, T, Bsz)(emb_table.reshape(V * D), ids_t)
    tvb = _make_tc_head(T, 2, Vout, D, Bsz)(xt3, W, b.reshape(Vout, 1))
    return jnp.transpose(tvb, (2, 0, 1))


# speedup vs baseline: 1.0319x; 1.0012x over previous
import functools

import jax
import jax.numpy as jnp
from jax import lax
from jax.experimental import pallas as pl
from jax.experimental.pallas import tpu as pltpu
from jax.experimental.pallas import tpu_sc as plsc


@functools.lru_cache(maxsize=None)
def _make_sc_lookup(V, D, T, B):
    info = plsc.get_sparse_core_info()
    NC, NS, L = info.num_cores, info.num_subcores, info.num_lanes
    NW = NC * NS
    assert D == L and B % (NW * L) == 0
    bw = B // NW
    n_h = bw // L

    mesh = plsc.VectorSubcoreMesh(core_axis_name="c", subcore_axis_name="s")

    @functools.partial(
        pl.kernel,
        out_type=jax.ShapeDtypeStruct((T, D, B), jnp.float32),
        mesh=mesh,
        compiler_params=pltpu.CompilerParams(
            use_tc_tiling_on_sc=False, needs_layout_passes=False
        ),
        scratch_types=[
            pltpu.VMEM((V * D,), jnp.float32),
            pltpu.VMEM((T, bw), jnp.int32),
            pltpu.VMEM((T, D, bw), jnp.float32),
        ],
    )
    def lookup(emb_hbm, ids_hbm, xt_hbm, emb_v, idx_v, tbuf):
        wid = lax.axis_index("s") * NC + lax.axis_index("c")
        bcol = wid * bw
        pltpu.sync_copy(emb_hbm, emb_v)
        pltpu.sync_copy(ids_hbm.at[:, pl.ds(bcol, bw)], idx_v)

        @plsc.parallel_loop(0, T, unroll=2)
        def tok(t):
            for h in range(n_h):
                idx_g = idx_v[t, pl.ds(h * L, L)]
                for d in range(D):
                    tbuf[t, d, pl.ds(h * L, L)] = plsc.load_gather(
                        emb_v, [idx_g * D + d]
                    )
        pltpu.sync_copy(tbuf, xt_hbm.at[:, :, pl.ds(bcol, bw)])

    return lookup


@functools.lru_cache(maxsize=None)
def _make_tc_head(T, BT, V, D, B):
    assert T % BT == 0

    def body(xt_ref, w_ref, b_ref, out_ref):
        for j in range(BT):
            out_ref[j] = (
                lax.dot_general(
                    w_ref[...], xt_ref[j],
                    dimension_numbers=(((0,), (0,)), ((), ())),
                    preferred_element_type=jnp.float32,
                )
                + b_ref[...]
            )

    return pl.pallas_call(
        body,
        grid=(T // BT,),
        in_specs=[
            pl.BlockSpec((BT, D, B), lambda i: (i, 0, 0)),
            pl.BlockSpec((D, V), lambda i: (0, 0)),
            pl.BlockSpec((V, 1), lambda i: (0, 0)),
        ],
        out_specs=pl.BlockSpec((BT, V, B), lambda i: (i, 0, 0)),
        out_shape=jax.ShapeDtypeStruct((T, V, B), jnp.float32),
        compiler_params=pltpu.CompilerParams(
            dimension_semantics=("parallel",),
        ),
    )


def kernel(input_ids, emb_table, W, b):
    Bsz, T = input_ids.shape
    V, D = emb_table.shape
    Vout = W.shape[1]

    ids_t = input_ids.astype(jnp.int32).T
    xt3 = _make_sc_lookup(V, D, T, Bsz)(emb_table.reshape(V * D), ids_t)
    tvb = _make_tc_head(T, 2, Vout, D, Bsz)(xt3, W, b.reshape(Vout, 1))
    return jnp.transpose(tvb, (2, 0, 1))
